# baseline (speedup 1.0000x reference)
import functools

import jax
import jax.numpy as jnp
from jax import lax
from jax.experimental import pallas as pl
from jax.experimental.pallas import tpu as pltpu
from jax.experimental.pallas import tpu_sc as plsc

_INFO = plsc.get_sparse_core_info()
_NC, _NS = _INFO.num_cores, _INFO.num_subcores
_NW = _NC * _NS

_CHUNK = 128


def _sc_gather(tok3, table, n_rows, d):
    cpw = tok3.shape[1]
    rows_per_w = cpw * _CHUNK
    mesh = plsc.VectorSubcoreMesh(core_axis_name="c", subcore_axis_name="s")

    @functools.partial(
        pl.kernel,
        out_type=jax.ShapeDtypeStruct((n_rows, d), jnp.float32),
        mesh=mesh,
        scratch_types=[
            pltpu.VMEM((cpw, _CHUNK), jnp.int32),
            pltpu.VMEM((_CHUNK, d), jnp.float32),
            pltpu.VMEM((_CHUNK, d), jnp.float32),
            pltpu.SemaphoreType.DMA,
            pltpu.SemaphoreType.DMA,
        ],
        compiler_params=pltpu.CompilerParams(use_tc_tiling_on_sc=False),
    )
    def gather_kernel(tok_hbm, table_hbm, out_hbm, idx_v, buf0, buf1, sem0, sem1):
        wid = lax.axis_index("s") * _NC + lax.axis_index("c")
        base = wid * rows_per_w
        pltpu.sync_copy(tok_hbm.at[wid], idx_v)

        bufs = (buf0, buf1)
        sems = (sem0, sem1)

        def start(c, p):
            pltpu.async_copy(table_hbm.at[idx_v.at[c]], bufs[p], sems[p])

        def drain(c, p):
            pltpu.make_async_copy(table_hbm.at[idx_v.at[c]], bufs[p], sems[p]).wait()
            pltpu.sync_copy(bufs[p], out_hbm.at[pl.ds(base + c * _CHUNK, _CHUNK)])

        start(0, 0)
        start(1, 1)

        @pl.loop(0, cpw // 2 - 1)
        def _(g):
            c = g * 2
            for p in range(2):
                drain(c + p, p)
                start(c + p + 2, p)

        c = cpw - 2
        for p in range(2):
            drain(c + p, p)

    return gather_kernel


def _matmul_body(emb_ref, w_ref, b_ref, out_ref):
    out_ref[...] = (
        lax.dot_general(
            emb_ref[...], w_ref[...],
            dimension_numbers=(((1,), (1,)), ((), ())),
            preferred_element_type=jnp.float32,
        )
        + b_ref[...]
    )


def _tc_matmul(emb, W, b2, blk):
    n, d = emb.shape
    grid = n // blk
    return pl.pallas_call(
        _matmul_body,
        grid=(grid,),
        in_specs=[
            pl.BlockSpec((blk, d), lambda i: (i, 0)),
            pl.BlockSpec((d, d), lambda i: (0, 0)),
            pl.BlockSpec((1, d), lambda i: (0, 0)),
        ],
        out_specs=pl.BlockSpec((blk, d), lambda i: (i, 0)),
        out_shape=jax.ShapeDtypeStruct((n, d), jnp.float32),
        compiler_params=pltpu.CompilerParams(
            dimension_semantics=("arbitrary",),
        ),
    )(emb, W, b2)


def kernel(text_tokens, embedding, W, b):
    batch, t = text_tokens.shape
    d = embedding.shape[1]
    n = batch * t
    assert n % (_NW * _CHUNK) == 0
    cpw = n // (_NW * _CHUNK)
    tok3 = text_tokens.reshape(_NW, cpw, _CHUNK).astype(jnp.int32)
    emb = _sc_gather(tok3, embedding, n, d)(tok3, embedding)
    out = _tc_matmul(emb, W, b.reshape(1, d), blk=2048)
    return out.reshape(batch, t, d)


# speedup vs baseline: 1.6888x; 1.6888x over previous
import functools

import jax
import jax.numpy as jnp
from jax import lax
from jax.experimental import pallas as pl
from jax.experimental.pallas import tpu as pltpu
from jax.experimental.pallas import tpu_sc as plsc

_INFO = plsc.get_sparse_core_info()
_NC, _NS = _INFO.num_cores, _INFO.num_subcores
_NW = _NC * _NS

_CHUNK = 128


def _sc_gather(t_steps, b_sz, d, vocab):
    half = b_sz // 2
    mesh = plsc.VectorSubcoreMesh(core_axis_name="c", subcore_axis_name="s")

    @functools.partial(
        pl.kernel,
        out_type=jax.ShapeDtypeStruct((t_steps, half, 2 * d), jnp.float32),
        mesh=mesh,
        scratch_types=[
            pltpu.VMEM((t_steps, _CHUNK), jnp.int32),
            pltpu.VMEM((_CHUNK, d), jnp.float32),
            pltpu.VMEM((_CHUNK, d), jnp.float32),
            pltpu.SemaphoreType.DMA,
            pltpu.SemaphoreType.DMA,
        ],
        compiler_params=pltpu.CompilerParams(use_tc_tiling_on_sc=False),
    )
    def gather_kernel(tok_hbm, table_hbm, out_hbm, idx_v, buf0, buf1, sem0, sem1):
        wid = lax.axis_index("s") * _NC + lax.axis_index("c")
        j0 = (wid % (half // _CHUNK)) * _CHUNK
        l0 = (wid // (half // _CHUNK)) * d
        pltpu.sync_copy(tok_hbm.at[wid], idx_v)

        bufs = (buf0, buf1)
        sems = (sem0, sem1)

        def start(t, p):
            pltpu.async_copy(table_hbm.at[idx_v.at[t]], bufs[p], sems[p])

        def drain(t, p):
            pltpu.make_async_copy(table_hbm.at[idx_v.at[t]], bufs[p], sems[p]).wait()
            pltpu.sync_copy(
                bufs[p], out_hbm.at[t, pl.ds(j0, _CHUNK), pl.ds(l0, d)]
            )

        start(0, 0)
        start(1, 1)

        @pl.loop(0, t_steps // 2 - 1)
        def _(g):
            t = g * 2
            for p in range(2):
                drain(t + p, p)
                start(t + p + 2, p)

        t = t_steps - 2
        for p in range(2):
            drain(t + p, p)

    return gather_kernel


def _matmul_body(emb_ref, w_ref, b_ref, out_ref):
    x = emb_ref[0]
    d2 = w_ref.shape[0]
    d = d2 // 2
    half = x.shape[0]
    z = lax.dot_general(
        w_ref[...], x,
        dimension_numbers=(((1,), (1,)), ((), ())),
        preferred_element_type=jnp.float32,
    )
    bias = b_ref[...]
    out_ref[0, :, 0:half] = z[0:d] + bias
    out_ref[0, :, half : 2 * half] = z[d:d2] + bias


def _tc_matmul(emb2, W2, b2, t_steps, b_sz, d):
    half = b_sz // 2
    return pl.pallas_call(
        _matmul_body,
        grid=(t_steps,),
        in_specs=[
            pl.BlockSpec((1, half, 2 * d), lambda i: (i, 0, 0)),
            pl.BlockSpec((2 * d, 2 * d), lambda i: (0, 0)),
            pl.BlockSpec((d, 1), lambda i: (0, 0)),
        ],
        out_specs=pl.BlockSpec((1, d, b_sz), lambda i: (i, 0, 0)),
        out_shape=jax.ShapeDtypeStruct((t_steps, d, b_sz), jnp.float32),
        compiler_params=pltpu.CompilerParams(
            dimension_semantics=("arbitrary",),
        ),
    )(emb2, W2, b2)


def kernel(text_tokens, embedding, W, b):
    b_sz, t_steps = text_tokens.shape
    d = embedding.shape[1]
    vocab = embedding.shape[0]
    assert b_sz % (2 * _CHUNK) == 0 and b_sz // _CHUNK == _NW

    tok_w = (
        text_tokens.astype(jnp.int32).T.reshape(t_steps, _NW, _CHUNK)
        .transpose(1, 0, 2)
    )
    emb2 = _sc_gather(t_steps, b_sz, d, vocab)(tok_w, embedding)

    W2 = jnp.zeros((2 * d, 2 * d), W.dtype).at[:d, :d].set(W).at[d:, d:].set(W)
    out_t = _tc_matmul(emb2, W2, b.reshape(d, 1), t_steps, b_sz, d)
    return jnp.transpose(out_t, (2, 0, 1))


# speedup vs baseline: 1.8558x; 1.0989x over previous
import functools

import jax
import jax.numpy as jnp
from jax import lax
from jax.experimental import pallas as pl
from jax.experimental.pallas import tpu as pltpu
from jax.experimental.pallas import tpu_sc as plsc

_INFO = plsc.get_sparse_core_info()
_NC, _NS = _INFO.num_cores, _INFO.num_subcores
_NW = _NC * _NS

_CHUNK = 128


def _sc_gather(t_steps, b_sz, d, vocab):
    half = b_sz // 2
    mesh = plsc.VectorSubcoreMesh(core_axis_name="c", subcore_axis_name="s")

    @functools.partial(
        pl.kernel,
        out_type=jax.ShapeDtypeStruct((t_steps, half, 2 * d), jnp.float32),
        mesh=mesh,
        scratch_types=[
            pltpu.VMEM((t_steps, _CHUNK), jnp.int32),
            pltpu.VMEM((_CHUNK, d), jnp.float32),
            pltpu.VMEM((_CHUNK, d), jnp.float32),
            pltpu.SemaphoreType.DMA,
            pltpu.SemaphoreType.DMA,
        ],
        compiler_params=pltpu.CompilerParams(use_tc_tiling_on_sc=False),
    )
    def gather_kernel(tok_hbm, table_hbm, out_hbm, idx_v, buf0, buf1, sem0, sem1):
        wid = lax.axis_index("s") * _NC + lax.axis_index("c")
        j0 = (wid % (half // _CHUNK)) * _CHUNK
        l0 = (wid // (half // _CHUNK)) * d
        pltpu.sync_copy(tok_hbm.at[wid], idx_v)

        bufs = (buf0, buf1)
        sems = (sem0, sem1)

        def start(t, p):
            pltpu.async_copy(table_hbm.at[idx_v.at[t]], bufs[p], sems[p])

        def drain(t, p):
            pltpu.make_async_copy(table_hbm.at[idx_v.at[t]], bufs[p], sems[p]).wait()
            pltpu.sync_copy(
                bufs[p], out_hbm.at[t, pl.ds(j0, _CHUNK), pl.ds(l0, d)]
            )

        start(0, 0)
        start(1, 1)

        @pl.loop(0, t_steps // 2 - 1)
        def _(g):
            t = g * 2
            for p in range(2):
                drain(t + p, p)
                start(t + p + 2, p)

        t = t_steps - 2
        for p in range(2):
            drain(t + p, p)

    return gather_kernel


def _matmul_body(emb_ref, w_ref, b_ref, out_ref):
    d2 = w_ref.shape[0]
    d = d2 // 2
    nt = emb_ref.shape[0]
    half = emb_ref.shape[1]
    bias = b_ref[...]
    for t in range(nt):
        x = emb_ref[t].astype(jnp.bfloat16)
        z = lax.dot_general(
            w_ref[...], x,
            dimension_numbers=(((1,), (1,)), ((), ())),
            preferred_element_type=jnp.float32,
        )
        out_ref[t, :, 0:half] = z[0:d] + bias
        out_ref[t, :, half : 2 * half] = z[d:d2] + bias


def _tc_matmul(emb2, W2, b2, t_steps, b_sz, d, nt=4):
    half = b_sz // 2
    return pl.pallas_call(
        _matmul_body,
        grid=(t_steps // nt,),
        in_specs=[
            pl.BlockSpec((nt, half, 2 * d), lambda i: (i, 0, 0)),
            pl.BlockSpec((2 * d, 2 * d), lambda i: (0, 0)),
            pl.BlockSpec((d, 1), lambda i: (0, 0)),
        ],
        out_specs=pl.BlockSpec((nt, d, b_sz), lambda i: (i, 0, 0)),
        out_shape=jax.ShapeDtypeStruct((t_steps, d, b_sz), jnp.float32),
        compiler_params=pltpu.CompilerParams(
            dimension_semantics=("arbitrary",),
        ),
    )(emb2, W2, b2)


def kernel(text_tokens, embedding, W, b):
    b_sz, t_steps = text_tokens.shape
    d = embedding.shape[1]
    vocab = embedding.shape[0]
    assert b_sz % (2 * _CHUNK) == 0 and b_sz // _CHUNK == _NW

    tok_w = (
        text_tokens.astype(jnp.int32).T.reshape(t_steps, _NW, _CHUNK)
        .transpose(1, 0, 2)
    )
    emb2 = _sc_gather(t_steps, b_sz, d, vocab)(tok_w, embedding)

    W2 = (
        jnp.zeros((2 * d, 2 * d), jnp.bfloat16)
        .at[:d, :d].set(W.astype(jnp.bfloat16))
        .at[d:, d:].set(W.astype(jnp.bfloat16))
    )
    out_t = _tc_matmul(emb2, W2, b.reshape(d, 1), t_steps, b_sz, d)
    return jnp.transpose(out_t, (2, 0, 1))


# speedup vs baseline: 2.2519x; 1.2134x over previous
import functools

import jax
import jax.numpy as jnp
from jax import lax
from jax.experimental import pallas as pl
from jax.experimental.pallas import tpu as pltpu
from jax.experimental.pallas import tpu_sc as plsc

_INFO = plsc.get_sparse_core_info()
_NC, _NS = _INFO.num_cores, _INFO.num_subcores
_NW = _NC * _NS

_CHUNK = 128


def _prep_body(a_ref, w_ref, bias_ref, out_ref):
    d = w_ref.shape[0]
    blk = a_ref.shape[1]
    bias = bias_ref[...]
    z = lax.dot_general(
        a_ref[...].astype(jnp.bfloat16), w_ref[...],
        dimension_numbers=(((0,), (1,)), ((), ())),
        preferred_element_type=jnp.float32,
    )
    z = z + bias
    z3 = z.reshape(blk // 2, 2, d)
    out_ref[:, 0:d] = z3[:, 0, :]
    out_ref[:, d : 2 * d] = z3[:, 1, :]


def _tc_prep(embT, Wb, bias2, vocab, d, blk=12800):
    grid = (vocab + blk - 1) // blk
    return pl.pallas_call(
        _prep_body,
        grid=(grid,),
        in_specs=[
            pl.BlockSpec((d, blk), lambda i: (0, i)),
            pl.BlockSpec((d, d), lambda i: (0, 0)),
            pl.BlockSpec((1, d), lambda i: (0, 0)),
        ],
        out_specs=pl.BlockSpec((blk // 2, 2 * d), lambda i: (i, 0)),
        out_shape=jax.ShapeDtypeStruct((vocab // 2, 2 * d), jnp.float32),
        compiler_params=pltpu.CompilerParams(
            dimension_semantics=("arbitrary",),
        ),
    )(embT, Wb, bias2)


def _sc_gather(t_steps, b_sz, d, vocab):
    half = b_sz // 2
    mesh = plsc.VectorSubcoreMesh(core_axis_name="c", subcore_axis_name="s")

    @functools.partial(
        pl.kernel,
        out_type=jax.ShapeDtypeStruct((t_steps, half, 2 * d), jnp.float32),
        mesh=mesh,
        scratch_types=[
            pltpu.VMEM((t_steps, _CHUNK), jnp.int32),
            pltpu.VMEM((_CHUNK, d), jnp.float32),
            pltpu.VMEM((_CHUNK, d), jnp.float32),
            pltpu.SemaphoreType.DMA,
            pltpu.SemaphoreType.DMA,
        ],
        compiler_params=pltpu.CompilerParams(use_tc_tiling_on_sc=False),
    )
    def gather_kernel(tok_hbm, table_hbm, out_hbm, idx_v, buf0, buf1, sem0, sem1):
        wid = lax.axis_index("s") * _NC + lax.axis_index("c")
        j0 = (wid % (half // _CHUNK)) * _CHUNK
        l0 = (wid // (half // _CHUNK)) * d
        pltpu.sync_copy(tok_hbm.at[wid], idx_v)

        bufs = (buf0, buf1)
        sems = (sem0, sem1)

        def start(t, p):
            pltpu.async_copy(table_hbm.at[idx_v.at[t]], bufs[p], sems[p])

        def drain(t, p):
            pltpu.make_async_copy(table_hbm.at[idx_v.at[t]], bufs[p], sems[p]).wait()
            pltpu.sync_copy(
                bufs[p], out_hbm.at[t, pl.ds(j0, _CHUNK), pl.ds(l0, d)]
            )

        start(0, 0)
        start(1, 1)

        @pl.loop(0, t_steps // 2 - 1)
        def _(g):
            t = g * 2
            for p in range(2):
                drain(t + p, p)
                start(t + p + 2, p)

        t = t_steps - 2
        for p in range(2):
            drain(t + p, p)

    return gather_kernel


def _transpose_body(emb_ref, i_ref, out_ref):
    d2 = i_ref.shape[0]
    d = d2 // 2
    nt = emb_ref.shape[0]
    half = emb_ref.shape[1]
    for t in range(nt):
        x = emb_ref[t].astype(jnp.bfloat16)
        z = lax.dot_general(
            i_ref[...], x,
            dimension_numbers=(((1,), (1,)), ((), ())),
            preferred_element_type=jnp.float32,
        )
        out_ref[t, :, 0:half] = z[0:d]
        out_ref[t, :, half : 2 * half] = z[d:d2]


def _tc_transpose(emb2, I2, t_steps, b_sz, d, nt=4):
    half = b_sz // 2
    return pl.pallas_call(
        _transpose_body,
        grid=(t_steps // nt,),
        in_specs=[
            pl.BlockSpec((nt, half, 2 * d), lambda i: (i, 0, 0)),
            pl.BlockSpec((2 * d, 2 * d), lambda i: (0, 0)),
        ],
        out_specs=pl.BlockSpec((nt, d, b_sz), lambda i: (i, 0, 0)),
        out_shape=jax.ShapeDtypeStruct((t_steps, d, b_sz), jnp.float32),
        compiler_params=pltpu.CompilerParams(
            dimension_semantics=("arbitrary",),
        ),
    )(emb2, I2)


def kernel(text_tokens, embedding, W, b):
    b_sz, t_steps = text_tokens.shape
    vocab, d = embedding.shape
    assert b_sz % (2 * _CHUNK) == 0 and b_sz // _CHUNK == _NW

    embT = embedding.T
    TT = _tc_prep(embT, W.astype(jnp.bfloat16), b.reshape(1, d), vocab, d)
    table_rm = TT.reshape(vocab, d)

    tok_w = (
        text_tokens.astype(jnp.int32).T.reshape(t_steps, _NW, _CHUNK)
        .transpose(1, 0, 2)
    )

    emb2 = _sc_gather(t_steps, b_sz, d, vocab)(tok_w, table_rm)

    I2 = jnp.eye(2 * d, dtype=jnp.bfloat16)
    out_t = _tc_transpose(emb2, I2, t_steps, b_sz, d)
    return jnp.transpose(out_t, (2, 0, 1))


# speedup vs baseline: 3.2218x; 1.4307x over previous
import functools

import jax
import jax.numpy as jnp
from jax import lax
from jax.experimental import pallas as pl
from jax.experimental.pallas import tpu as pltpu
from jax.experimental.pallas import tpu_sc as plsc

_INFO = plsc.get_sparse_core_info()
_NC, _NS = _INFO.num_cores, _INFO.num_subcores
_NW = _NC * _NS

_CHUNK = 128


def _prep_body(a_ref, w_ref, bias_ref, out_ref):
    d = w_ref.shape[0]
    blk = a_ref.shape[1]
    bias = bias_ref[...]
    z = lax.dot_general(
        a_ref[...].astype(jnp.bfloat16), w_ref[...],
        dimension_numbers=(((0,), (1,)), ((), ())),
        preferred_element_type=jnp.float32,
    )
    z = z + bias
    out_ref[:, 0:d] = z[0 : blk // 2]
    out_ref[:, d : 2 * d] = z[blk // 2 : blk]


def _tc_prep(embT, Wb, bias2, vocab, d, blk=12800):
    grid = (vocab + blk - 1) // blk
    return pl.pallas_call(
        _prep_body,
        grid=(grid,),
        in_specs=[
            pl.BlockSpec((d, blk), lambda i: (0, i)),
            pl.BlockSpec((d, d), lambda i: (0, 0)),
            pl.BlockSpec((1, d), lambda i: (0, 0)),
        ],
        out_specs=pl.BlockSpec((blk // 2, 2 * d), lambda i: (i, 0)),
        out_shape=jax.ShapeDtypeStruct((grid * blk // 2, 2 * d), jnp.float32),
        compiler_params=pltpu.CompilerParams(
            dimension_semantics=("arbitrary",),
        ),
    )(embT, Wb, bias2)


def _sc_gather(t_steps, b_sz, d, vocab):
    half = b_sz // 2
    mesh = plsc.VectorSubcoreMesh(core_axis_name="c", subcore_axis_name="s")

    @functools.partial(
        pl.kernel,
        out_type=jax.ShapeDtypeStruct((t_steps, half, 2 * d), jnp.float32),
        mesh=mesh,
        scratch_types=[
            pltpu.VMEM((t_steps, _CHUNK), jnp.int32),
            pltpu.VMEM((_CHUNK, d), jnp.float32),
            pltpu.VMEM((_CHUNK, d), jnp.float32),
            pltpu.SemaphoreType.DMA,
            pltpu.SemaphoreType.DMA,
        ],
        compiler_params=pltpu.CompilerParams(use_tc_tiling_on_sc=False),
    )
    def gather_kernel(tok_hbm, table_hbm, out_hbm, idx_v, buf0, buf1, sem0, sem1):
        wid = lax.axis_index("s") * _NC + lax.axis_index("c")
        j0 = (wid % (half // _CHUNK)) * _CHUNK
        l0 = (wid // (half // _CHUNK)) * d
        pltpu.sync_copy(tok_hbm.at[wid], idx_v)

        bufs = (buf0, buf1)
        sems = (sem0, sem1)

        def start(t, p):
            pltpu.async_copy(table_hbm.at[idx_v.at[t]], bufs[p], sems[p])

        def drain(t, p):
            pltpu.make_async_copy(table_hbm.at[idx_v.at[t]], bufs[p], sems[p]).wait()
            pltpu.sync_copy(
                bufs[p], out_hbm.at[t, pl.ds(j0, _CHUNK), pl.ds(l0, d)]
            )

        start(0, 0)
        start(1, 1)

        @pl.loop(0, t_steps // 2 - 1)
        def _(g):
            t = g * 2
            for p in range(2):
                drain(t + p, p)
                start(t + p + 2, p)

        t = t_steps - 2
        for p in range(2):
            drain(t + p, p)

    return gather_kernel


def _transpose_body(emb_ref, i_ref, out_ref):
    d2 = i_ref.shape[0]
    d = d2 // 2
    nt = emb_ref.shape[0]
    half = emb_ref.shape[1]
    for t in range(nt):
        x = emb_ref[t].astype(jnp.bfloat16)
        z = lax.dot_general(
            i_ref[...], x,
            dimension_numbers=(((1,), (1,)), ((), ())),
            preferred_element_type=jnp.float32,
        )
        out_ref[t, :, 0:half] = z[0:d]
        out_ref[t, :, half : 2 * half] = z[d:d2]


def _tc_transpose(emb2, I2, t_steps, b_sz, d, nt=4):
    half = b_sz // 2
    return pl.pallas_call(
        _transpose_body,
        grid=(t_steps // nt,),
        in_specs=[
            pl.BlockSpec((nt, half, 2 * d), lambda i: (i, 0, 0)),
            pl.BlockSpec((2 * d, 2 * d), lambda i: (0, 0)),
        ],
        out_specs=pl.BlockSpec((nt, d, b_sz), lambda i: (i, 0, 0)),
        out_shape=jax.ShapeDtypeStruct((t_steps, d, b_sz), jnp.float32),
        compiler_params=pltpu.CompilerParams(
            dimension_semantics=("arbitrary",),
        ),
    )(emb2, I2)


def kernel(text_tokens, embedding, W, b):
    b_sz, t_steps = text_tokens.shape
    vocab, d = embedding.shape
    assert b_sz % (2 * _CHUNK) == 0 and b_sz // _CHUNK == _NW

    blk = 12800
    embT = embedding.T
    TT = _tc_prep(embT, W.astype(jnp.bfloat16), b.reshape(1, d), vocab, d, blk)
    table_rm = TT.reshape(TT.shape[0] * 2, d)

    tok = text_tokens.astype(jnp.int32)
    jj = tok % blk
    s = (tok - jj) + 2 * (jj % (blk // 2)) + jj // (blk // 2)
    tok_w = s.T.reshape(t_steps, _NW, _CHUNK).transpose(1, 0, 2)

    emb2 = _sc_gather(t_steps, b_sz, d, vocab)(tok_w, table_rm)

    I2 = jnp.eye(2 * d, dtype=jnp.bfloat16)
    out_t = _tc_transpose(emb2, I2, t_steps, b_sz, d)
    return jnp.transpose(out_t, (2, 0, 1))


# speedup vs baseline: 3.2550x; 1.0103x over previous
import functools

import jax
import jax.numpy as jnp
from jax import lax
from jax.experimental import pallas as pl
from jax.experimental.pallas import tpu as pltpu
from jax.experimental.pallas import tpu_sc as plsc

_INFO = plsc.get_sparse_core_info()
_NC, _NS = _INFO.num_cores, _INFO.num_subcores
_NW = _NC * _NS

_CHUNK = 128


def _prep_body(a_ref, w_ref, bias_ref, out_ref):
    d = w_ref.shape[0]
    blk = a_ref.shape[1]
    bias = bias_ref[...]
    z = lax.dot_general(
        a_ref[...].astype(jnp.bfloat16), w_ref[...],
        dimension_numbers=(((0,), (1,)), ((), ())),
        preferred_element_type=jnp.float32,
    )
    z = z + bias
    out_ref[:, 0:d] = z[0 : blk // 2]
    out_ref[:, d : 2 * d] = z[blk // 2 : blk]


def _tc_prep(embT, Wb, bias2, vocab, d, blk=12800):
    grid = (vocab + blk - 1) // blk
    return pl.pallas_call(
        _prep_body,
        grid=(grid,),
        in_specs=[
            pl.BlockSpec((d, blk), lambda i: (0, i)),
            pl.BlockSpec((d, d), lambda i: (0, 0)),
            pl.BlockSpec((1, d), lambda i: (0, 0)),
        ],
        out_specs=pl.BlockSpec((blk // 2, 2 * d), lambda i: (i, 0)),
        out_shape=jax.ShapeDtypeStruct((grid * blk // 2, 2 * d), jnp.float32),
        compiler_params=pltpu.CompilerParams(
            dimension_semantics=("arbitrary",),
        ),
    )(embT, Wb, bias2)


def _sc_gather(t_steps, b_sz, d, vocab):
    half = b_sz // 2
    mesh = plsc.VectorSubcoreMesh(core_axis_name="c", subcore_axis_name="s")

    @functools.partial(
        pl.kernel,
        out_type=jax.ShapeDtypeStruct((t_steps, half, 2 * d), jnp.float32),
        mesh=mesh,
        scratch_types=[
            pltpu.VMEM((t_steps, _CHUNK), jnp.int32),
            pltpu.VMEM((_CHUNK, d), jnp.float32),
            pltpu.VMEM((_CHUNK, d), jnp.float32),
            pltpu.SemaphoreType.DMA,
            pltpu.SemaphoreType.DMA,
        ],
        compiler_params=pltpu.CompilerParams(use_tc_tiling_on_sc=False),
    )
    def gather_kernel(tok_hbm, table_hbm, out_hbm, idx_v, buf0, buf1, sem0, sem1):
        wid = lax.axis_index("s") * _NC + lax.axis_index("c")
        j0 = (wid % (half // _CHUNK)) * _CHUNK
        l0 = (wid // (half // _CHUNK)) * d
        pltpu.sync_copy(tok_hbm.at[wid], idx_v)

        bufs = (buf0, buf1)
        sems = (sem0, sem1)

        def start(t, p):
            pltpu.async_copy(table_hbm.at[idx_v.at[t]], bufs[p], sems[p])

        def drain(t, p):
            pltpu.make_async_copy(table_hbm.at[idx_v.at[t]], bufs[p], sems[p]).wait()
            pltpu.sync_copy(
                bufs[p], out_hbm.at[t, pl.ds(j0, _CHUNK), pl.ds(l0, d)]
            )

        start(0, 0)
        start(1, 1)

        @pl.loop(0, t_steps // 2 - 1)
        def _(g):
            t = g * 2
            for p in range(2):
                drain(t + p, p)
                start(t + p + 2, p)

        t = t_steps - 2
        for p in range(2):
            drain(t + p, p)

    return gather_kernel


def _transpose_body(emb_ref, i_ref, *rest):
    out_ref = rest[-1]
    d2 = i_ref.shape[0]
    d = d2 // 2
    nt = emb_ref.shape[0]
    half = emb_ref.shape[1]
    for t in range(nt):
        x = emb_ref[t].astype(jnp.bfloat16)
        z = lax.dot_general(
            i_ref[...], x,
            dimension_numbers=(((1,), (1,)), ((), ())),
            preferred_element_type=jnp.float32,
        )
        out_ref[t, :, 0:half] = z[0:d]
        out_ref[t, :, half : 2 * half] = z[d:d2]


def _tc_transpose(emb2, I2, out_prev, c, tc, t_steps, b_sz, d, nt=4):
    half = b_sz // 2
    c0 = c * (tc // nt)
    in_specs = [
        pl.BlockSpec((nt, half, 2 * d), lambda i: (i, 0, 0)),
        pl.BlockSpec((2 * d, 2 * d), lambda i: (0, 0)),
    ]
    args = [emb2, I2]
    aliases = {}
    if out_prev is not None:
        in_specs.append(pl.BlockSpec(memory_space=pl.ANY))
        args.append(out_prev)
        aliases = {2: 0}
    return pl.pallas_call(
        _transpose_body,
        grid=(tc // nt,),
        in_specs=in_specs,
        out_specs=pl.BlockSpec((nt, d, b_sz), lambda i, _c0=c0: (i + _c0, 0, 0)),
        out_shape=jax.ShapeDtypeStruct((t_steps, d, b_sz), jnp.float32),
        input_output_aliases=aliases,
        compiler_params=pltpu.CompilerParams(
            dimension_semantics=("arbitrary",),
        ),
    )(*args)


def kernel(text_tokens, embedding, W, b):
    b_sz, t_steps = text_tokens.shape
    vocab, d = embedding.shape
    assert b_sz % (2 * _CHUNK) == 0 and b_sz // _CHUNK == _NW

    blk = 12800
    embT = embedding.T
    TT = _tc_prep(embT, W.astype(jnp.bfloat16), b.reshape(1, d), vocab, d, blk)
    table_rm = TT.reshape(TT.shape[0] * 2, d)

    tok = text_tokens.astype(jnp.int32)
    jj = tok % blk
    s = (tok - jj) + 2 * (jj % (blk // 2)) + jj // (blk // 2)
    tok_w = s.T.reshape(t_steps, _NW, _CHUNK).transpose(1, 0, 2)

    n_chunk = 2
    tc = t_steps // n_chunk
    embs = [
        _sc_gather(tc, b_sz, d, vocab)(tok_w[:, c * tc : (c + 1) * tc], table_rm)
        for c in range(n_chunk)
    ]
    I2 = jnp.eye(2 * d, dtype=jnp.bfloat16)
    out_t = None
    for c in range(n_chunk):
        out_t = _tc_transpose(embs[c], I2, out_t, c, tc, t_steps, b_sz, d)
    return jnp.transpose(out_t, (2, 0, 1))


# speedup vs baseline: 3.4006x; 1.0447x over previous
import functools

import jax
import jax.numpy as jnp
from jax import lax
from jax.experimental import pallas as pl
from jax.experimental.pallas import tpu as pltpu
from jax.experimental.pallas import tpu_sc as plsc

_INFO = plsc.get_sparse_core_info()
_NC, _NS = _INFO.num_cores, _INFO.num_subcores
_NW = _NC * _NS

_CHUNK = 128


def _prep_body(a_ref, w_ref, bias_ref, out_ref):
    d = w_ref.shape[0]
    blk = a_ref.shape[1]
    bias = bias_ref[...]
    z = lax.dot_general(
        a_ref[...].astype(jnp.bfloat16), w_ref[...],
        dimension_numbers=(((0,), (1,)), ((), ())),
        preferred_element_type=jnp.float32,
    )
    z = z + bias
    out_ref[:, 0:d] = z[0 : blk // 2]
    out_ref[:, d : 2 * d] = z[blk // 2 : blk]


def _tc_prep(embT, Wb, bias2, vocab, d, blk=12800):
    grid = (vocab + blk - 1) // blk
    return pl.pallas_call(
        _prep_body,
        grid=(grid,),
        in_specs=[
            pl.BlockSpec((d, blk), lambda i: (0, i)),
            pl.BlockSpec((d, d), lambda i: (0, 0)),
            pl.BlockSpec((1, d), lambda i: (0, 0)),
        ],
        out_specs=pl.BlockSpec((blk // 2, 2 * d), lambda i: (i, 0)),
        out_shape=jax.ShapeDtypeStruct((grid * blk // 2, 2 * d), jnp.float32),
        compiler_params=pltpu.CompilerParams(
            dimension_semantics=("arbitrary",),
        ),
    )(embT, Wb, bias2)


def _sc_gather(t_steps, b_sz, d, vocab):
    half = b_sz // 2
    mesh = plsc.VectorSubcoreMesh(core_axis_name="c", subcore_axis_name="s")

    @functools.partial(
        pl.kernel,
        out_type=jax.ShapeDtypeStruct((t_steps, half, 2 * d), jnp.float32),
        mesh=mesh,
        scratch_types=[
            pltpu.VMEM((t_steps, _CHUNK), jnp.int32),
            pltpu.VMEM((_CHUNK, d), jnp.float32),
            pltpu.VMEM((_CHUNK, d), jnp.float32),
            pltpu.SemaphoreType.DMA,
            pltpu.SemaphoreType.DMA,
        ],
        compiler_params=pltpu.CompilerParams(use_tc_tiling_on_sc=False),
    )
    def gather_kernel(tok_hbm, table_hbm, out_hbm, idx_v, buf0, buf1, sem0, sem1):
        wid = lax.axis_index("s") * _NC + lax.axis_index("c")
        j0 = (wid % (half // _CHUNK)) * _CHUNK
        l0 = (wid // (half // _CHUNK)) * d
        pltpu.sync_copy(tok_hbm.at[wid], idx_v)

        bufs = (buf0, buf1)
        sems = (sem0, sem1)

        def start(t, p):
            pltpu.async_copy(table_hbm.at[idx_v.at[t]], bufs[p], sems[p])

        def drain(t, p):
            pltpu.make_async_copy(table_hbm.at[idx_v.at[t]], bufs[p], sems[p]).wait()
            pltpu.sync_copy(
                bufs[p], out_hbm.at[t, pl.ds(j0, _CHUNK), pl.ds(l0, d)]
            )

        start(0, 0)
        start(1, 1)

        @pl.loop(0, t_steps // 2 - 1)
        def _(g):
            t = g * 2
            for p in range(2):
                drain(t + p, p)
                start(t + p + 2, p)

        t = t_steps - 2
        for p in range(2):
            drain(t + p, p)

    return gather_kernel


def _transpose_body(emb_ref, i_ref, *rest):
    out_ref = rest[-1]
    d2 = i_ref.shape[0]
    d = d2 // 2
    nt = emb_ref.shape[0]
    half = emb_ref.shape[1]
    for t in range(nt):
        x = emb_ref[t].astype(jnp.bfloat16)
        z = lax.dot_general(
            i_ref[...], x,
            dimension_numbers=(((1,), (1,)), ((), ())),
            preferred_element_type=jnp.float32,
        )
        out_ref[t, :, 0:half] = z[0:d]
        out_ref[t, :, half : 2 * half] = z[d:d2]


def _tc_transpose(emb2, I2, out_prev, c, tc, t_steps, b_sz, d, nt=4):
    half = b_sz // 2
    c0 = c * (tc // nt)
    in_specs = [
        pl.BlockSpec((nt, half, 2 * d), lambda i: (i, 0, 0)),
        pl.BlockSpec((2 * d, 2 * d), lambda i: (0, 0)),
    ]
    args = [emb2, I2]
    aliases = {}
    if out_prev is not None:
        in_specs.append(pl.BlockSpec(memory_space=pl.ANY))
        args.append(out_prev)
        aliases = {2: 0}
    return pl.pallas_call(
        _transpose_body,
        grid=(tc // nt,),
        in_specs=in_specs,
        out_specs=pl.BlockSpec((nt, d, b_sz), lambda i, _c0=c0: (i + _c0, 0, 0)),
        out_shape=jax.ShapeDtypeStruct((t_steps, d, b_sz), jnp.float32),
        input_output_aliases=aliases,
        compiler_params=pltpu.CompilerParams(
            dimension_semantics=("arbitrary",),
        ),
    )(*args)


def kernel(text_tokens, embedding, W, b):
    b_sz, t_steps = text_tokens.shape
    vocab, d = embedding.shape
    assert b_sz % (2 * _CHUNK) == 0 and b_sz // _CHUNK == _NW

    blk = 25600
    embT = embedding.T
    TT = _tc_prep(embT, W.astype(jnp.bfloat16), b.reshape(1, d), vocab, d, blk)
    table_rm = TT.reshape(TT.shape[0] * 2, d)

    tok = text_tokens.astype(jnp.int32)
    jj = tok % blk
    s = (tok - jj) + 2 * (jj % (blk // 2)) + jj // (blk // 2)
    tok_w = s.T.reshape(t_steps, _NW, _CHUNK).transpose(1, 0, 2)

    n_chunk = 4
    tc = t_steps // n_chunk
    embs = [
        _sc_gather(tc, b_sz, d, vocab)(tok_w[:, c * tc : (c + 1) * tc], table_rm)
        for c in range(n_chunk)
    ]
    I2 = jnp.eye(2 * d, dtype=jnp.bfloat16)
    out_t = None
    for c in range(n_chunk):
        out_t = _tc_transpose(embs[c], I2, out_t, c, tc, t_steps, b_sz, d, nt=5)
    return jnp.transpose(out_t, (2, 0, 1))
